# baseline (speedup 1.0000x reference)
import functools

import jax
import jax.numpy as jnp
from jax import lax
from jax.experimental import pallas as pl
from jax.experimental.pallas import tpu as pltpu
from jax.experimental.pallas import tpu_sc as plsc

T = 2048
D = 768
E = 8
K = 2
F = 384

BTG = 256
NBR = (T * K) // BTG + E
NP = NBR * BTG
CHUNK = T // 32
SUB = 16


def _router_kernel(x_ref, wg_ref, pos1_ref, pos2_ref, w1x_ref, w2x_ref,
                   meta_ref):
    x = x_ref[...]
    logits = jnp.dot(x, wg_ref[...], preferred_element_type=jnp.float32)
    scores = jax.nn.sigmoid(logits)
    eids = lax.broadcasted_iota(jnp.int32, (T, E), 1)
    idx1 = jnp.argmax(scores, axis=1)
    v1 = jnp.max(scores, axis=1)
    oh1 = (eids == idx1[:, None]).astype(jnp.float32)
    masked = jnp.where(oh1 > 0, -jnp.inf, scores)
    idx2 = jnp.argmax(masked, axis=1)
    v2 = jnp.max(masked, axis=1)
    oh2 = (eids == idx2[:, None]).astype(jnp.float32)

    denom = v1 + v2 + 1e-20
    w1x_ref[...] = jnp.broadcast_to((v1 / denom)[:, None], (T, 16))
    w2x_ref[...] = jnp.broadcast_to((v2 / denom)[:, None], (T, 16))

    ohsum = oh1 + oh2
    g = jnp.sum(ohsum, axis=0, keepdims=True)
    pc = jnp.floor((g + (BTG - 1)) * (1.0 / BTG))
    pcB = pc * BTG
    ei = lax.broadcasted_iota(jnp.int32, (E, E), 0)
    ej = lax.broadcasted_iota(jnp.int32, (E, E), 1)
    su8 = (ei < ej).astype(jnp.float32)
    iu8 = (ei <= ej).astype(jnp.float32)
    po = jnp.dot(pcB, su8, preferred_element_type=jnp.float32)
    cblk = jnp.dot(pc, iu8, preferred_element_type=jnp.float32)

    b128 = lax.broadcasted_iota(jnp.int32, (1, 128), 1).astype(jnp.float32)
    bexp = jnp.zeros((1, 128), jnp.float32)
    g_at = jnp.zeros((1, 128), jnp.float32)
    po_at = jnp.zeros((1, 128), jnp.float32)
    for e in range(E):
        bexp = bexp + (b128 >= cblk[0, e]).astype(jnp.float32)
    for e in range(E):
        sel = (bexp == e).astype(jnp.float32)
        g_at = g_at + sel * g[0, e]
        po_at = po_at + sel * po[0, e]
    vcnt = jnp.clip(g_at - (BTG * b128 - po_at), 0.0, float(BTG))
    rows8 = lax.broadcasted_iota(jnp.int32, (8, 128), 0)
    meta = jnp.where(rows8 == 0, jnp.broadcast_to(bexp, (8, 128)),
                     jnp.where(rows8 == 1, jnp.broadcast_to(vcnt, (8, 128)),
                               0.0))
    meta_ref[...] = meta.astype(jnp.int32)

    nch = T // 256
    li = lax.broadcasted_iota(jnp.int32, (256, 256), 0)
    lj = lax.broadcasted_iota(jnp.int32, (256, 256), 1)
    l256 = (li > lj).astype(jnp.float32)
    carry = jnp.zeros((1, E), jnp.float32)
    p1_parts = []
    p2_parts = []
    for c in range(nch):
        sl = slice(c * 256, (c + 1) * 256)
        ch = ohsum[sl]
        cc = jnp.dot(l256, ch, preferred_element_type=jnp.float32) + carry
        dest = po + cc
        p1_parts.append(jnp.sum(oh1[sl] * dest, axis=1, keepdims=True))
        p2_parts.append(jnp.sum(oh2[sl] * dest, axis=1, keepdims=True))
        carry = carry + jnp.sum(ch, axis=0, keepdims=True)
    pos1 = jnp.concatenate(p1_parts, axis=0)
    pos2 = jnp.concatenate(p2_parts, axis=0)
    pos1_ref[...] = jnp.broadcast_to(pos1, (T, 8)).astype(jnp.int32)
    pos2_ref[...] = jnp.broadcast_to(pos2, (T, 8)).astype(jnp.int32)


def _run_router(x, Wg):
    return pl.pallas_call(
        _router_kernel,
        grid=(1,),
        in_specs=[
            pl.BlockSpec((T, D), lambda i: (0, 0)),
            pl.BlockSpec((D, E), lambda i: (0, 0)),
        ],
        out_specs=[
            pl.BlockSpec((T, 8), lambda i: (0, 0)),
            pl.BlockSpec((T, 8), lambda i: (0, 0)),
            pl.BlockSpec((T, 16), lambda i: (0, 0)),
            pl.BlockSpec((T, 16), lambda i: (0, 0)),
            pl.BlockSpec((8, 128), lambda i: (0, 0)),
        ],
        out_shape=[
            jax.ShapeDtypeStruct((T, 8), jnp.int32),
            jax.ShapeDtypeStruct((T, 8), jnp.int32),
            jax.ShapeDtypeStruct((T, 16), jnp.float32),
            jax.ShapeDtypeStruct((T, 16), jnp.float32),
            jax.ShapeDtypeStruct((8, 128), jnp.int32),
        ],
    )(x, Wg)


def _shared_kernel(x_ref, wsg_ref, wsu_ref, wsd_ref, y_ref):
    xq = x_ref[...]
    g = jnp.dot(xq, wsg_ref[...], preferred_element_type=jnp.float32)
    u = jnp.dot(xq, wsu_ref[...], preferred_element_type=jnp.float32)
    y_ref[...] = jnp.dot(jax.nn.silu(g) * u, wsd_ref[...],
                         preferred_element_type=jnp.float32)


def _run_shared(x, Wsg, Wsu, Wsd):
    return pl.pallas_call(
        _shared_kernel,
        grid=(T // BTG,),
        in_specs=[
            pl.BlockSpec((BTG, D), lambda i: (i, 0)),
            pl.BlockSpec((D, F), lambda i: (0, 0)),
            pl.BlockSpec((D, F), lambda i: (0, 0)),
            pl.BlockSpec((F, D), lambda i: (0, 0)),
        ],
        out_specs=pl.BlockSpec((BTG, D), lambda i: (i, 0)),
        out_shape=jax.ShapeDtypeStruct((T, D), jnp.float32),
    )(x, Wsg, Wsu, Wsd)


def _make_scatter():
    mesh = plsc.VectorSubcoreMesh(core_axis_name="c", subcore_axis_name="s")

    @functools.partial(
        pl.kernel, mesh=mesh,
        out_type=jax.ShapeDtypeStruct((NP, D), jnp.float32),
        scratch_types=[
            pltpu.VMEM((CHUNK, D), jnp.float32),
            pltpu.VMEM((CHUNK,), jnp.int32),
            pltpu.VMEM((CHUNK,), jnp.int32),
            pltpu.SemaphoreType.DMA,
            pltpu.SemaphoreType.DMA,
            pltpu.SemaphoreType.DMA,
        ],
    )
    def scatter(x_hbm, pos1_hbm, pos2_hbm, xs_hbm, rows_v, p1_v, p2_v,
                s1, s2, s3):
        wid = lax.axis_index("s") * 2 + lax.axis_index("c")
        base = wid * CHUNK
        cx = pltpu.async_copy(x_hbm.at[pl.ds(base, CHUNK)], rows_v, s1)
        cp1 = pltpu.async_copy(pos1_hbm.at[pl.ds(base, CHUNK)], p1_v, s2)
        cp2 = pltpu.async_copy(pos2_hbm.at[pl.ds(base, CHUNK)], p2_v, s3)
        cx.wait()
        cp1.wait()
        cp2.wait()
        c1 = pltpu.async_copy(rows_v, xs_hbm.at[p1_v], s1)
        c2 = pltpu.async_copy(rows_v, xs_hbm.at[p2_v], s2)
        c1.wait()
        c2.wait()

    return scatter


def _ffn_kernel(bexp_ref, vcnt_ref, xs_ref, w1g_ref, w1u_ref, w2_ref, y_ref):
    b = pl.program_id(0)
    vc = vcnt_ref[b]

    @pl.when(vc > 0)
    def _():
        eid = bexp_ref[b]
        xq = xs_ref[...]
        g = jnp.dot(xq, w1g_ref[eid], preferred_element_type=jnp.float32)
        u = jnp.dot(xq, w1u_ref[eid], preferred_element_type=jnp.float32)
        rows = lax.broadcasted_iota(jnp.int32, (BTG, 1), 0)
        inter = jnp.where(rows < vc, jax.nn.silu(g) * u, 0.0)
        y_ref[...] = jnp.dot(inter, w2_ref[eid],
                             preferred_element_type=jnp.float32)


def _run_ffn(bexp, vcnt, xs, W1g, W1u, W2):
    grid_spec = pltpu.PrefetchScalarGridSpec(
        num_scalar_prefetch=2,
        grid=(NBR,),
        in_specs=[
            pl.BlockSpec((BTG, D), lambda b, be, vcn: (b, 0)),
            pl.BlockSpec((E, D, F), lambda b, be, vcn: (0, 0, 0)),
            pl.BlockSpec((E, D, F), lambda b, be, vcn: (0, 0, 0)),
            pl.BlockSpec((E, F, D), lambda b, be, vcn: (0, 0, 0)),
        ],
        out_specs=pl.BlockSpec((BTG, D), lambda b, be, vcn: (b, 0)),
    )
    return pl.pallas_call(
        _ffn_kernel,
        grid_spec=grid_spec,
        out_shape=jax.ShapeDtypeStruct((NP, D), jnp.float32),
    )(bexp, vcnt, xs, W1g, W1u, W2)


def _make_combine():
    mesh = plsc.VectorSubcoreMesh(core_axis_name="c", subcore_axis_name="s")
    NSUB = CHUNK // SUB

    @functools.partial(
        pl.kernel, mesh=mesh,
        out_type=jax.ShapeDtypeStruct((T, D), jnp.float32),
        scratch_types=[
            pltpu.VMEM((2, SUB, D), jnp.float32),
            pltpu.VMEM((2, SUB, D), jnp.float32),
            pltpu.VMEM((2, SUB, D), jnp.float32),
            pltpu.VMEM((CHUNK,), jnp.int32),
            pltpu.VMEM((CHUNK,), jnp.int32),
            pltpu.VMEM((CHUNK, 16), jnp.float32),
            pltpu.VMEM((CHUNK, 16), jnp.float32),
            pltpu.SemaphoreType.DMA,
            pltpu.SemaphoreType.DMA,
            pltpu.SemaphoreType.DMA,
            pltpu.SemaphoreType.DMA,
        ],
    )
    def combine(y_hbm, ysh_hbm, pos1_hbm, pos2_hbm, w1x_hbm, w2x_hbm, out_hbm,
                y1_v, y2_v, ysh_v, p1_v, p2_v, w1_v, w2_v, s1, s2, s3, so):
        wid = lax.axis_index("s") * 2 + lax.axis_index("c")
        cbase = wid * CHUNK
        cp1 = pltpu.async_copy(pos1_hbm.at[pl.ds(cbase, CHUNK)], p1_v, s1)
        cp2 = pltpu.async_copy(pos2_hbm.at[pl.ds(cbase, CHUNK)], p2_v, s2)
        cw1 = pltpu.async_copy(w1x_hbm.at[pl.ds(cbase, CHUNK)], w1_v, s3)
        cw2 = pltpu.async_copy(w2x_hbm.at[pl.ds(cbase, CHUNK)], w2_v, so)
        cp1.wait()
        cp2.wait()
        cw1.wait()
        cw2.wait()

        def issue(s, slot):
            base = cbase + s * SUB
            g1 = pltpu.async_copy(y_hbm.at[p1_v.at[pl.ds(s * SUB, SUB)]],
                                  y1_v.at[slot], s1)
            g2 = pltpu.async_copy(y_hbm.at[p2_v.at[pl.ds(s * SUB, SUB)]],
                                  y2_v.at[slot], s2)
            g3 = pltpu.async_copy(ysh_hbm.at[pl.ds(base, SUB)],
                                  ysh_v.at[slot], s3)
            return g1, g2, g3

        pend = issue(0, 0)
        for s in range(NSUB):
            slot = s % 2
            for c in pend:
                c.wait()
            if s + 1 < NSUB:
                pend = issue(s + 1, 1 - slot)

            def body(r4, _):
                for k in range(4):
                    r = r4 * 4 + k
                    wa = w1_v[s * SUB + r]
                    wb = w2_v[s * SUB + r]
                    for j in range(D // 16):
                        cs = pl.ds(j * 16, 16)
                        y1_v[slot, r, cs] = (wa * y1_v[slot, r, cs]
                                             + wb * y2_v[slot, r, cs]
                                             + ysh_v[slot, r, cs])
                return 0

            lax.fori_loop(0, SUB // 4, body, 0)
            pltpu.sync_copy(y1_v.at[slot],
                            out_hbm.at[pl.ds(cbase + s * SUB, SUB)])

    return combine


_make_scatter = functools.cache(_make_scatter)
_make_combine = functools.cache(_make_combine)


@jax.jit
def kernel(hidden_states, Wg, W1g, W1u, W2, Wsg, Wsu, Wsd):
    x = hidden_states
    pos1_2d, pos2_2d, w1x, w2x, meta = _run_router(x, Wg)
    pos1 = pos1_2d[:, 0]
    pos2 = pos2_2d[:, 0]
    bexp = jnp.minimum(meta[0, :NBR], E - 1)
    vcnt = meta[1, :NBR]

    xs = _make_scatter()(x, pos1, pos2)
    ysh = _run_shared(x, Wsg, Wsu, Wsd)
    y = _run_ffn(bexp, vcnt, xs, W1g, W1u, W2)
    return _make_combine()(y, ysh, pos1, pos2, w1x, w2x)


# speedup vs baseline: 1.7546x; 1.7546x over previous
import functools

import jax
import jax.numpy as jnp
from jax.experimental import pallas as pl

T = 2048
D = 768
E = 8
K = 2
F = 384
FS = 384

BT = 1024


def _moe_block_kernel(x_ref, wg_ref, w1g_ref, w1u_ref, w2_ref,
                      wsg_ref, wsu_ref, wsd_ref, out_ref):
    xf = x_ref[...]
    xb = xf.astype(jnp.bfloat16)

    logits = jnp.dot(xf, wg_ref[...], preferred_element_type=jnp.float32)
    scores = jax.nn.sigmoid(logits)
    eids = jax.lax.broadcasted_iota(jnp.int32, (BT, E), 1)
    idx1 = jnp.argmax(scores, axis=1)
    v1 = jnp.max(scores, axis=1)
    oh1 = eids == idx1[:, None]
    masked = jnp.where(oh1, -jnp.inf, scores)
    idx2 = jnp.argmax(masked, axis=1)
    v2 = jnp.max(masked, axis=1)
    oh2 = eids == idx2[:, None]
    denom = v1 + v2 + 1e-20
    combine = (oh1 * v1[:, None] + oh2 * v2[:, None]) / denom[:, None]

    sg = jnp.dot(xb, wsg_ref[...], preferred_element_type=jnp.float32)
    su = jnp.dot(xb, wsu_ref[...], preferred_element_type=jnp.float32)
    inter_s = (jax.nn.silu(sg) * su).astype(jnp.bfloat16)
    acc = jnp.dot(inter_s, wsd_ref[...], preferred_element_type=jnp.float32)

    for e in range(E):
        g = jnp.dot(xb, w1g_ref[e], preferred_element_type=jnp.float32)
        u = jnp.dot(xb, w1u_ref[e], preferred_element_type=jnp.float32)
        inter = (jax.nn.silu(g) * u).astype(jnp.bfloat16)
        acc = acc + jnp.dot(inter, w2_ref[e],
                            preferred_element_type=jnp.float32) * combine[:, e:e + 1]

    out_ref[...] = acc


@jax.jit
def kernel(hidden_states, Wg, W1g, W1u, W2, Wsg, Wsu, Wsd):
    bf = jnp.bfloat16
    grid = (T // BT,)
    return pl.pallas_call(
        _moe_block_kernel,
        grid=grid,
        in_specs=[
            pl.BlockSpec((BT, D), lambda i: (i, 0)),
            pl.BlockSpec((D, E), lambda i: (0, 0)),
            pl.BlockSpec((E, D, F), lambda i: (0, 0, 0)),
            pl.BlockSpec((E, D, F), lambda i: (0, 0, 0)),
            pl.BlockSpec((E, F, D), lambda i: (0, 0, 0)),
            pl.BlockSpec((D, FS), lambda i: (0, 0)),
            pl.BlockSpec((D, FS), lambda i: (0, 0)),
            pl.BlockSpec((FS, D), lambda i: (0, 0)),
        ],
        out_specs=pl.BlockSpec((BT, D), lambda i: (i, 0)),
        out_shape=jax.ShapeDtypeStruct((T, D), jnp.float32),
    )(hidden_states, Wg, W1g, W1u, W2, Wsg, Wsu, Wsd)
